# baseline (speedup 1.0000x reference)
import functools

import jax
import jax.numpy as jnp
from jax import lax
from jax.experimental import pallas as pl
from jax.experimental.pallas import tpu as pltpu
from jax.experimental.pallas import tpu_sc as plsc


def _gather_kernel(B, D, NC, NW):
    b_per_w = B // NW
    mesh = plsc.VectorSubcoreMesh(core_axis_name="c", subcore_axis_name="s")

    @functools.partial(
        pl.kernel,
        mesh=mesh,
        out_type=jax.ShapeDtypeStruct((B, D), jnp.float32),
        scratch_types=[
            pltpu.VMEM((b_per_w,), jnp.int32),
            pltpu.SemaphoreType.DMA,
        ],
    )
    def k(idx_hbm, table_hbm, out_hbm, idx_v, sem):
        wid = lax.axis_index("s") * NC + lax.axis_index("c")
        base = wid * b_per_w
        pltpu.sync_copy(idx_hbm.at[pl.ds(base, b_per_w)], idx_v)

        def issue(t, _):
            j0 = t * 16
            v = idx_v[pl.ds(j0, 16)]
            for l in range(16):
                pltpu.async_copy(table_hbm.at[v[l]], out_hbm.at[base + j0 + l], sem)
            return _

        lax.fori_loop(0, b_per_w // 16, issue, 0)

        def drain(j, _):
            pltpu.make_async_copy(table_hbm.at[0], out_hbm.at[base], sem).wait()
            return _

        lax.fori_loop(0, b_per_w, drain, 0)

    return k


def kernel(domains, table):
    B, = domains.shape
    V, D = table.shape
    info = plsc.get_sparse_core_info()
    NC, NS = info.num_cores, info.num_subcores
    NW = NC * NS
    k = _gather_kernel(B, D, NC, NW)
    return k(domains, table)


# speedup vs baseline: 9.2674x; 9.2674x over previous
import functools

import jax
import jax.numpy as jnp
from jax import lax
from jax.experimental import pallas as pl
from jax.experimental.pallas import tpu as pltpu
from jax.experimental.pallas import tpu_sc as plsc

_CH = 4096
_L = 16


def _gather_kernel(B, V, D, NC, NW):
    n_ch = B // _CH
    mesh = plsc.VectorSubcoreMesh(core_axis_name="c", subcore_axis_name="s")

    @functools.partial(
        pl.kernel,
        mesh=mesh,
        out_type=jax.ShapeDtypeStruct((D, B), jnp.float32),
        compiler_params=pltpu.CompilerParams(needs_layout_passes=False),
        scratch_types=[
            pltpu.VMEM((V,), jnp.float32),
            pltpu.VMEM((_CH,), jnp.int32),
            pltpu.VMEM((_CH,), jnp.float32),
            pltpu.SemaphoreType.DMA,
        ],
    )
    def k(idx_hbm, tab_hbm, out_hbm, col_v, idx_v, out_v, sem):
        c = lax.axis_index("s") * NC + lax.axis_index("c")
        col_copy = pltpu.async_copy(tab_hbm.at[c], col_v, sem)
        for ch in range(n_ch):
            pltpu.sync_copy(idx_hbm.at[pl.ds(ch * _CH, _CH)], idx_v)
            if ch == 0:
                col_copy.wait()

            @plsc.parallel_loop(0, _CH, step=_L, unroll=8)
            def _(t):
                out_v[pl.ds(t, _L)] = plsc.load_gather(
                    col_v, [idx_v[pl.ds(t, _L)]]
                )

            pltpu.sync_copy(out_v, out_hbm.at[c, pl.ds(ch * _CH, _CH)])

    return k


def kernel(domains, table):
    B, = domains.shape
    V, D = table.shape
    info = plsc.get_sparse_core_info()
    NC, NS = info.num_cores, info.num_subcores
    NW = NC * NS
    k = _gather_kernel(B, V, D, NC, NW)
    return k(domains, table.T).T


# speedup vs baseline: 10.5315x; 1.1364x over previous
import functools

import jax
import jax.numpy as jnp
from jax import lax
from jax.experimental import pallas as pl
from jax.experimental.pallas import tpu as pltpu
from jax.experimental.pallas import tpu_sc as plsc

_CH = 4096
_L = 16


def _gather_kernel(B, V, D, NC, NW):
    n_ch = B // _CH
    mesh = plsc.VectorSubcoreMesh(core_axis_name="c", subcore_axis_name="s")

    @functools.partial(
        pl.kernel,
        mesh=mesh,
        out_type=jax.ShapeDtypeStruct((D, B), jnp.float32),
        compiler_params=pltpu.CompilerParams(needs_layout_passes=False),
        scratch_types=[
            pltpu.VMEM((V,), jnp.float32),
            pltpu.VMEM((n_ch, _CH), jnp.int32),
            pltpu.VMEM((2, _CH), jnp.float32),
            [pltpu.SemaphoreType.DMA] * 2,
            [pltpu.SemaphoreType.DMA] * n_ch,
            [pltpu.SemaphoreType.DMA] * 2,
        ],
    )
    def k(idx_hbm, tab_hbm, out_hbm, col_v, idx_v, out_v, csems, isems, osems):
        c = lax.axis_index("s") * NC + lax.axis_index("c")
        idx_cps = [
            pltpu.async_copy(
                idx_hbm.at[pl.ds(ch * _CH, _CH)], idx_v.at[ch], isems[ch]
            )
            for ch in range(n_ch)
        ]
        pltpu.async_copy(tab_hbm.at[c], col_v, csems[0]).wait()
        wb = [None, None]
        for ch in range(n_ch):
            idx_cps[ch].wait()
            if wb[ch % 2] is not None:
                wb[ch % 2].wait()

            @plsc.parallel_loop(0, _CH, step=_L, unroll=8)
            def _(t):
                out_v[ch % 2, pl.ds(t, _L)] = plsc.load_gather(
                    col_v, [idx_v[ch, pl.ds(t, _L)]]
                )

            wb[ch % 2] = pltpu.async_copy(
                out_v.at[ch % 2],
                out_hbm.at[c, pl.ds(ch * _CH, _CH)],
                osems[ch % 2],
            )
        for cp in wb:
            cp.wait()

    return k


def kernel(domains, table):
    B, = domains.shape
    V, D = table.shape
    info = plsc.get_sparse_core_info()
    NC, NS = info.num_cores, info.num_subcores
    NW = NC * NS
    k = _gather_kernel(B, V, D, NC, NW)
    return k(domains, table.T).T
